# baseline (speedup 1.0000x reference)
import functools

import jax
import jax.numpy as jnp
from jax import lax
from jax.experimental import pallas as pl
from jax.experimental.pallas import tpu as pltpu
from jax.experimental.pallas import tpu_sc as plsc

N = 10000
D = 128
E = 320000
NC = 2
NS = 16
NW = NC * NS
PER_W = E // NW
CH = 64
NCH = PER_W // CH
REM = PER_W - NCH * CH
NBUF = 4
STG = 3
ZSTRIPE = 640
N_ACC = NS * ZSTRIPE
OUT_STRIPE = 624
OUT_TAIL = N - NS * OUT_STRIPE


def _sc_aggregate(x, adj):
    mesh = plsc.VectorSubcoreMesh(core_axis_name="c", subcore_axis_name="s")

    @functools.partial(
        pl.kernel,
        out_type=jax.ShapeDtypeStruct((NC, N, D), jnp.float32),
        mesh=mesh,
        scratch_types=(
            [pltpu.VMEM((PER_W,), jnp.int32)]
            + [pltpu.VMEM((CH,), jnp.int32)] * NBUF
            + [pltpu.VMEM((CH, D), jnp.float32)] * NBUF
            + [pltpu.VMEM((REM,), jnp.int32),
               pltpu.VMEM((REM, D), jnp.float32),
               pltpu.VMEM_SHARED((N_ACC, D), jnp.float32)]
            + [pltpu.SemaphoreType.DMA] * (3 * NBUF + 3)
        ),
    )
    def k(x_hbm, adj_hbm, out_hbm, *refs):
        sidx_all = refs[0]
        didx = refs[1:1 + NBUF]
        rows = refs[1 + NBUF:1 + 2 * NBUF]
        didx_r, rows_r, acc = refs[1 + 2 * NBUF:4 + 2 * NBUF]
        sems = refs[4 + 2 * NBUF:]
        sem_d = sems[0:NBUF]
        sem_g = sems[NBUF:2 * NBUF]
        sem_s = sems[2 * NBUF:3 * NBUF]
        sem_x, sem_dr, sem_gr = sems[3 * NBUF:]

        cid = lax.axis_index("c")
        sid = lax.axis_index("s")
        wid = sid * NC + cid
        base = wid * PER_W


        cp_s = pltpu.async_copy(adj_hbm.at[pl.ds(base, PER_W)], sidx_all, sem_x)
        cp_d = [pltpu.async_copy(adj_hbm.at[pl.ds(E + base + b * CH, CH)],
                                 didx[b], sem_d[b]) for b in range(STG)]
        cp_dr = pltpu.async_copy(adj_hbm.at[pl.ds(E + base + NCH * CH, REM)],
                                 didx_r, sem_dr)

        zero16 = jnp.zeros((16,), jnp.float32)

        @pl.loop(0, CH)
        def _(r):
            @pl.loop(0, D, step=16)
            def _(c2):
                rows[0][r, pl.ds(c2, 16)] = zero16

        @pl.loop(0, ZSTRIPE, step=CH)
        def _(i):
            pltpu.sync_copy(rows[0], acc.at[pl.ds(sid * ZSTRIPE + i, CH)])

        cp_s.wait()
        cp_g = [pltpu.async_copy(
            x_hbm.at[sidx_all.at[pl.ds(b * CH, CH)]], rows[b], sem_g[b])
            for b in range(STG)]
        cp_gr = pltpu.async_copy(
            x_hbm.at[sidx_all.at[pl.ds(NCH * CH, REM)]], rows_r, sem_gr)

        all_cp_d = cp_d + [pltpu.make_async_copy(
            adj_hbm.at[pl.ds(E + base + b * CH, CH)], didx[b], sem_d[b])
            for b in range(STG, NBUF)]
        all_cp_g = cp_g + [pltpu.make_async_copy(
            x_hbm.at[sidx_all.at[pl.ds(b * CH, CH)]], rows[b], sem_g[b])
            for b in range(STG, NBUF)]
        cp_sc = [pltpu.make_async_copy(rows[b], acc.at[didx[b]], sem_s[b])
                 for b in range(NBUF)]

        plsc.subcore_barrier()

        @pl.loop(0, NCH, step=NBUF)
        def _(i):
            for b in range(NBUF):
                j = i + b
                pb = (b + STG) % NBUF

                @pl.when(j >= NBUF - STG)
                def _():
                    cp_sc[pb].wait()

                @pl.when(j + STG < NCH)
                def _():
                    pltpu.async_copy(
                        adj_hbm.at[pl.ds(E + base + (j + STG) * CH, CH)],
                        didx[pb], sem_d[pb])
                    pltpu.async_copy(
                        x_hbm.at[sidx_all.at[pl.ds((j + STG) * CH, CH)]],
                        rows[pb], sem_g[pb])

                all_cp_g[b].wait()
                all_cp_d[b].wait()
                pltpu.async_copy(rows[b], acc.at[didx[b]], sem_s[b], add=True)

        for t in range(NBUF - STG):
            cp_sc[(NCH - 1 - t) % NBUF].wait()
        cp_gr.wait()
        cp_dr.wait()
        pltpu.sync_copy(rows_r, acc.at[didx_r], add=True)

        plsc.subcore_barrier()

        pltpu.sync_copy(acc.at[pl.ds(sid * OUT_STRIPE, OUT_STRIPE)],
                        out_hbm.at[cid].at[pl.ds(sid * OUT_STRIPE, OUT_STRIPE)])

        @pl.when(sid == 0)
        def _():
            pltpu.sync_copy(acc.at[pl.ds(NS * OUT_STRIPE, OUT_TAIL)],
                            out_hbm.at[cid].at[pl.ds(NS * OUT_STRIPE, OUT_TAIL)])

    return k(x, adj.reshape(-1))


BLK = 1000


def _tc_mlp(x, agg, W1, b1r, W2, b2r, eps_row):
    def body(x_ref, a_ref, w1_ref, b1_ref, w2_ref, b2_ref, e_ref, o_ref):
        h = (1.0 + e_ref[...]) * x_ref[...] + a_ref[0] + a_ref[1]
        h = jnp.dot(h, w1_ref[...], preferred_element_type=jnp.float32)
        h = jnp.maximum(h + b1_ref[...], 0.0)
        o = jnp.dot(h, w2_ref[...], preferred_element_type=jnp.float32)
        o_ref[...] = o + b2_ref[...]

    return pl.pallas_call(
        body,
        grid=(N // BLK,),
        in_specs=[
            pl.BlockSpec((BLK, D), lambda i: (i, 0)),
            pl.BlockSpec((NC, BLK, D), lambda i: (0, i, 0)),
            pl.BlockSpec((D, D), lambda i: (0, 0)),
            pl.BlockSpec((1, D), lambda i: (0, 0)),
            pl.BlockSpec((D, D), lambda i: (0, 0)),
            pl.BlockSpec((1, D), lambda i: (0, 0)),
            pl.BlockSpec((1, D), lambda i: (0, 0)),
        ],
        out_specs=pl.BlockSpec((BLK, D), lambda i: (i, 0)),
        out_shape=jax.ShapeDtypeStruct((N, D), jnp.float32),
    )(x, agg, W1, b1r, W2, b2r, eps_row)


def kernel(x, adj, W1, b1, W2, b2, eps):
    adj32 = adj.astype(jnp.int32)
    agg = _sc_aggregate(x, adj32)
    b1r = b1.reshape(1, D)
    b2r = b2.reshape(1, D)
    eps_row = jnp.broadcast_to(eps, (1, D)).astype(jnp.float32)
    return _tc_mlp(x, agg, W1, b1r, W2, b2r, eps_row)


# speedup vs baseline: 1.0305x; 1.0305x over previous
import functools

import jax
import jax.numpy as jnp
from jax import lax
from jax.experimental import pallas as pl
from jax.experimental.pallas import tpu as pltpu
from jax.experimental.pallas import tpu_sc as plsc

N = 10000
D = 128
E = 320000
NC = 2
NS = 16
NW = NC * NS
PER_W = E // NW
CH = 64
NCH = PER_W // CH
REM = PER_W - NCH * CH
NBUF = 4
STG = 3
ZSTRIPE = 640
N_ACC = NS * ZSTRIPE
OUT_STRIPE = 624
OUT_TAIL = N - NS * OUT_STRIPE


def _sc_aggregate(x, adj):
    mesh = plsc.VectorSubcoreMesh(core_axis_name="c", subcore_axis_name="s")

    @functools.partial(
        pl.kernel,
        out_type=jax.ShapeDtypeStruct((NC, N, D), jnp.float32),
        mesh=mesh,
        scratch_types=(
            [pltpu.VMEM((PER_W,), jnp.int32)]
            + [pltpu.VMEM((CH,), jnp.int32)] * NBUF
            + [pltpu.VMEM((CH, D), jnp.float32)] * NBUF
            + [pltpu.VMEM((REM,), jnp.int32),
               pltpu.VMEM((REM, D), jnp.float32),
               pltpu.VMEM_SHARED((N_ACC, D), jnp.float32)]
            + [pltpu.SemaphoreType.DMA] * (3 * NBUF + 3)
        ),
    )
    def k(x_hbm, adj_hbm, out_hbm, *refs):
        sidx_all = refs[0]
        didx = refs[1:1 + NBUF]
        rows = refs[1 + NBUF:1 + 2 * NBUF]
        didx_r, rows_r, acc = refs[1 + 2 * NBUF:4 + 2 * NBUF]
        sems = refs[4 + 2 * NBUF:]
        sem_d = sems[0:NBUF]
        sem_g = sems[NBUF:2 * NBUF]
        sem_s = sems[2 * NBUF:3 * NBUF]
        sem_x, sem_dr, sem_gr = sems[3 * NBUF:]

        cid = lax.axis_index("c")
        sid = lax.axis_index("s")
        wid = sid * NC + cid
        base = wid * PER_W


        cp_s = pltpu.async_copy(adj_hbm.at[pl.ds(base, PER_W)], sidx_all, sem_x)
        cp_d = [pltpu.async_copy(adj_hbm.at[pl.ds(E + base + b * CH, CH)],
                                 didx[b], sem_d[b]) for b in range(STG)]
        cp_dr = pltpu.async_copy(adj_hbm.at[pl.ds(E + base + NCH * CH, REM)],
                                 didx_r, sem_dr)

        cp_s.wait()
        cp_g = [pltpu.async_copy(
            x_hbm.at[sidx_all.at[pl.ds(b * CH, CH)]], rows[b], sem_g[b])
            for b in range(STG)]
        cp_gr = pltpu.async_copy(
            x_hbm.at[sidx_all.at[pl.ds(NCH * CH, REM)]], rows_r, sem_gr)

        zero16 = jnp.zeros((16,), jnp.float32)
        ztile = rows[NBUF - 1]

        @pl.loop(0, CH)
        def _(r):
            @pl.loop(0, D, step=16)
            def _(c2):
                ztile[r, pl.ds(c2, 16)] = zero16

        @pl.loop(0, ZSTRIPE, step=CH)
        def _(i):
            pltpu.sync_copy(ztile, acc.at[pl.ds(sid * ZSTRIPE + i, CH)])

        all_cp_d = cp_d + [pltpu.make_async_copy(
            adj_hbm.at[pl.ds(E + base + b * CH, CH)], didx[b], sem_d[b])
            for b in range(STG, NBUF)]
        all_cp_g = cp_g + [pltpu.make_async_copy(
            x_hbm.at[sidx_all.at[pl.ds(b * CH, CH)]], rows[b], sem_g[b])
            for b in range(STG, NBUF)]
        cp_sc = [pltpu.make_async_copy(rows[b], acc.at[didx[b]], sem_s[b])
                 for b in range(NBUF)]

        plsc.subcore_barrier()

        @pl.loop(0, NCH, step=NBUF)
        def _(i):
            for b in range(NBUF):
                j = i + b
                pb = (b + STG) % NBUF

                @pl.when(j >= NBUF - STG)
                def _():
                    cp_sc[pb].wait()

                @pl.when(j + STG < NCH)
                def _():
                    pltpu.async_copy(
                        adj_hbm.at[pl.ds(E + base + (j + STG) * CH, CH)],
                        didx[pb], sem_d[pb])
                    pltpu.async_copy(
                        x_hbm.at[sidx_all.at[pl.ds((j + STG) * CH, CH)]],
                        rows[pb], sem_g[pb])

                all_cp_g[b].wait()
                all_cp_d[b].wait()
                pltpu.async_copy(rows[b], acc.at[didx[b]], sem_s[b], add=True)

        for t in range(NBUF - STG):
            cp_sc[(NCH - 1 - t) % NBUF].wait()
        cp_gr.wait()
        cp_dr.wait()
        pltpu.sync_copy(rows_r, acc.at[didx_r], add=True)

        plsc.subcore_barrier()

        pltpu.sync_copy(acc.at[pl.ds(sid * OUT_STRIPE, OUT_STRIPE)],
                        out_hbm.at[cid].at[pl.ds(sid * OUT_STRIPE, OUT_STRIPE)])

        @pl.when(sid == 0)
        def _():
            pltpu.sync_copy(acc.at[pl.ds(NS * OUT_STRIPE, OUT_TAIL)],
                            out_hbm.at[cid].at[pl.ds(NS * OUT_STRIPE, OUT_TAIL)])

    return k(x, adj.reshape(-1))


BLK = 2000


def _tc_mlp(x, agg, W1, b1r, W2, b2r, eps_row):
    def body(x_ref, a_ref, w1_ref, b1_ref, w2_ref, b2_ref, e_ref, o_ref):
        h = (1.0 + e_ref[...]) * x_ref[...] + a_ref[0] + a_ref[1]
        h = jnp.dot(h, w1_ref[...], preferred_element_type=jnp.float32)
        h = jnp.maximum(h + b1_ref[...], 0.0)
        o = jnp.dot(h, w2_ref[...], preferred_element_type=jnp.float32)
        o_ref[...] = o + b2_ref[...]

    return pl.pallas_call(
        body,
        grid=(N // BLK,),
        in_specs=[
            pl.BlockSpec((BLK, D), lambda i: (i, 0)),
            pl.BlockSpec((NC, BLK, D), lambda i: (0, i, 0)),
            pl.BlockSpec((D, D), lambda i: (0, 0)),
            pl.BlockSpec((1, D), lambda i: (0, 0)),
            pl.BlockSpec((D, D), lambda i: (0, 0)),
            pl.BlockSpec((1, D), lambda i: (0, 0)),
            pl.BlockSpec((1, D), lambda i: (0, 0)),
        ],
        out_specs=pl.BlockSpec((BLK, D), lambda i: (i, 0)),
        out_shape=jax.ShapeDtypeStruct((N, D), jnp.float32),
    )(x, agg, W1, b1r, W2, b2r, eps_row)


def kernel(x, adj, W1, b1, W2, b2, eps):
    adj32 = adj.astype(jnp.int32)
    agg = _sc_aggregate(x, adj32)
    b1r = b1.reshape(1, D)
    b2r = b2.reshape(1, D)
    eps_row = jnp.broadcast_to(eps, (1, D)).astype(jnp.float32)
    return _tc_mlp(x, agg, W1, b1r, W2, b2r, eps_row)


# speedup vs baseline: 1.0316x; 1.0010x over previous
import functools

import jax
import jax.numpy as jnp
from jax import lax
from jax.experimental import pallas as pl
from jax.experimental.pallas import tpu as pltpu
from jax.experimental.pallas import tpu_sc as plsc

N = 10000
D = 128
E = 320000
NC = 2
NS = 16
NW = NC * NS
PER_W = E // NW
CH = 64
NCH = PER_W // CH
REM = PER_W - NCH * CH
NBUF = 4
STG = 3
ZSTRIPE = 640
N_ACC = NS * ZSTRIPE
OUT_STRIPE = 624
OUT_TAIL = N - NS * OUT_STRIPE


def _sc_aggregate(x, adj):
    mesh = plsc.VectorSubcoreMesh(core_axis_name="c", subcore_axis_name="s")

    @functools.partial(
        pl.kernel,
        out_type=jax.ShapeDtypeStruct((NC, N, D), jnp.float32),
        mesh=mesh,
        scratch_types=(
            [pltpu.VMEM((PER_W,), jnp.int32)]
            + [pltpu.VMEM((CH,), jnp.int32)] * NBUF
            + [pltpu.VMEM((CH, D), jnp.float32)] * NBUF
            + [pltpu.VMEM((REM,), jnp.int32),
               pltpu.VMEM((REM, D), jnp.float32),
               pltpu.VMEM_SHARED((N_ACC, D), jnp.float32)]
            + [pltpu.SemaphoreType.DMA] * (3 * NBUF + 3)
        ),
    )
    def k(x_hbm, adj_hbm, out_hbm, *refs):
        sidx_all = refs[0]
        didx = refs[1:1 + NBUF]
        rows = refs[1 + NBUF:1 + 2 * NBUF]
        didx_r, rows_r, acc = refs[1 + 2 * NBUF:4 + 2 * NBUF]
        sems = refs[4 + 2 * NBUF:]
        sem_d = sems[0:NBUF]
        sem_g = sems[NBUF:2 * NBUF]
        sem_s = sems[2 * NBUF:3 * NBUF]
        sem_x, sem_dr, sem_gr = sems[3 * NBUF:]

        cid = lax.axis_index("c")
        sid = lax.axis_index("s")
        wid = sid * NC + cid
        base = wid * PER_W


        cp_s = pltpu.async_copy(adj_hbm.at[pl.ds(base, PER_W)], sidx_all, sem_x)
        cp_d = [pltpu.async_copy(adj_hbm.at[pl.ds(E + base + b * CH, CH)],
                                 didx[b], sem_d[b]) for b in range(STG)]
        cp_dr = pltpu.async_copy(adj_hbm.at[pl.ds(E + base + NCH * CH, REM)],
                                 didx_r, sem_dr)

        cp_s.wait()
        cp_g = [pltpu.async_copy(
            x_hbm.at[sidx_all.at[pl.ds(b * CH, CH)]], rows[b], sem_g[b])
            for b in range(STG)]
        cp_gr = pltpu.async_copy(
            x_hbm.at[sidx_all.at[pl.ds(NCH * CH, REM)]], rows_r, sem_gr)

        zero16 = jnp.zeros((16,), jnp.float32)
        ztile = rows[NBUF - 1]

        @pl.loop(0, CH)
        def _(r):
            @pl.loop(0, D, step=16)
            def _(c2):
                ztile[r, pl.ds(c2, 16)] = zero16

        @pl.loop(0, ZSTRIPE, step=CH)
        def _(i):
            pltpu.sync_copy(ztile, acc.at[pl.ds(sid * ZSTRIPE + i, CH)])

        all_cp_d = cp_d + [pltpu.make_async_copy(
            adj_hbm.at[pl.ds(E + base + b * CH, CH)], didx[b], sem_d[b])
            for b in range(STG, NBUF)]
        all_cp_g = cp_g + [pltpu.make_async_copy(
            x_hbm.at[sidx_all.at[pl.ds(b * CH, CH)]], rows[b], sem_g[b])
            for b in range(STG, NBUF)]
        cp_sc = [pltpu.make_async_copy(rows[b], acc.at[didx[b]], sem_s[b])
                 for b in range(NBUF)]

        plsc.subcore_barrier()

        @pl.loop(0, NCH, step=NBUF)
        def _(i):
            for b in range(NBUF):
                j = i + b
                pb = (b + STG) % NBUF

                @pl.when(j >= NBUF - STG)
                def _():
                    cp_sc[pb].wait()

                @pl.when(j + STG < NCH)
                def _():
                    pltpu.async_copy(
                        adj_hbm.at[pl.ds(E + base + (j + STG) * CH, CH)],
                        didx[pb], sem_d[pb])
                    pltpu.async_copy(
                        x_hbm.at[sidx_all.at[pl.ds((j + STG) * CH, CH)]],
                        rows[pb], sem_g[pb])

                all_cp_g[b].wait()
                all_cp_d[b].wait()
                pltpu.async_copy(rows[b], acc.at[didx[b]], sem_s[b], add=True)

        for t in range(NBUF - STG):
            cp_sc[(NCH - 1 - t) % NBUF].wait()
        cp_gr.wait()
        cp_dr.wait()
        pltpu.sync_copy(rows_r, acc.at[didx_r], add=True)

        plsc.subcore_barrier()

        HS = OUT_STRIPE // 2
        cpo = [pltpu.async_copy(
            acc.at[pl.ds(sid * OUT_STRIPE + h * HS, HS)],
            out_hbm.at[cid].at[pl.ds(sid * OUT_STRIPE + h * HS, HS)],
            sem_g[h]) for h in range(2)]

        @pl.when(sid == 0)
        def _():
            pltpu.async_copy(acc.at[pl.ds(NS * OUT_STRIPE, OUT_TAIL)],
                             out_hbm.at[cid].at[pl.ds(NS * OUT_STRIPE, OUT_TAIL)],
                             sem_g[2])

        cpo[0].wait()
        cpo[1].wait()

        @pl.when(sid == 0)
        def _():
            pltpu.make_async_copy(
                acc.at[pl.ds(NS * OUT_STRIPE, OUT_TAIL)],
                out_hbm.at[cid].at[pl.ds(NS * OUT_STRIPE, OUT_TAIL)],
                sem_g[2]).wait()

    return k(x, adj.reshape(-1))


BLK = 2000


def _tc_mlp(x, agg, W1, b1r, W2, b2r, eps_row):
    def body(x_ref, a_ref, w1_ref, b1_ref, w2_ref, b2_ref, e_ref, o_ref):
        h = (1.0 + e_ref[...]) * x_ref[...] + a_ref[0] + a_ref[1]
        h = jnp.dot(h, w1_ref[...], preferred_element_type=jnp.float32)
        h = jnp.maximum(h + b1_ref[...], 0.0)
        o = jnp.dot(h, w2_ref[...], preferred_element_type=jnp.float32)
        o_ref[...] = o + b2_ref[...]

    return pl.pallas_call(
        body,
        grid=(N // BLK,),
        in_specs=[
            pl.BlockSpec((BLK, D), lambda i: (i, 0)),
            pl.BlockSpec((NC, BLK, D), lambda i: (0, i, 0)),
            pl.BlockSpec((D, D), lambda i: (0, 0)),
            pl.BlockSpec((1, D), lambda i: (0, 0)),
            pl.BlockSpec((D, D), lambda i: (0, 0)),
            pl.BlockSpec((1, D), lambda i: (0, 0)),
            pl.BlockSpec((1, D), lambda i: (0, 0)),
        ],
        out_specs=pl.BlockSpec((BLK, D), lambda i: (i, 0)),
        out_shape=jax.ShapeDtypeStruct((N, D), jnp.float32),
    )(x, agg, W1, b1r, W2, b2r, eps_row)


def kernel(x, adj, W1, b1, W2, b2, eps):
    adj32 = adj.astype(jnp.int32)
    agg = _sc_aggregate(x, adj32)
    b1r = b1.reshape(1, D)
    b2r = b2.reshape(1, D)
    eps_row = jnp.broadcast_to(eps, (1, D)).astype(jnp.float32)
    return _tc_mlp(x, agg, W1, b1r, W2, b2r, eps_row)
